# baseline (speedup 1.0000x reference)
import functools

import jax
import jax.numpy as jnp
from jax.experimental import pallas as pl

_EMBED_DIM = 64
_NUM_CODES = 1024
_TILE = 1152


def _vq_body(z_ref, w_ref, out_ref):
    zt = z_ref[...]
    zn = zt / jnp.maximum(
        jnp.sqrt(jnp.sum(zt * zt, axis=1, keepdims=True)), 1e-12)
    w = w_ref[...]
    wn = w / jnp.maximum(
        jnp.sqrt(jnp.sum(w * w, axis=1, keepdims=True)), 1e-12)
    scores = jax.lax.dot_general(
        zn, wn, (((1,), (1,)), ((), ())),
        preferred_element_type=jnp.float32)
    m = jnp.max(scores, axis=1, keepdims=True)
    ids = jax.lax.broadcasted_iota(jnp.int32, scores.shape, 1)
    idx = jnp.min(jnp.where(scores == m, ids, jnp.int32(2**30)), axis=1)
    onehot = (ids == idx[:, None]).astype(jnp.float32)
    out_ref[...] = jax.lax.dot_general(
        onehot, w, (((1,), (0,)), ((), ())),
        preferred_element_type=jnp.float32,
        precision=jax.lax.Precision.HIGHEST)


@jax.jit
def kernel(z, W):
    z2 = z.reshape(-1, _EMBED_DIM)
    n_tok = z2.shape[0]
    grid = (n_tok // _TILE,)
    zq = pl.pallas_call(
        _vq_body,
        grid=grid,
        in_specs=[
            pl.BlockSpec((_TILE, _EMBED_DIM), lambda i: (i, 0)),
            pl.BlockSpec((_NUM_CODES, _EMBED_DIM), lambda i: (0, 0)),
        ],
        out_specs=pl.BlockSpec((_TILE, _EMBED_DIM), lambda i: (i, 0)),
        out_shape=jax.ShapeDtypeStruct((n_tok, _EMBED_DIM), jnp.float32),
    )(z2, W)
    return zq.reshape(z.shape)


# speedup vs baseline: 1.1081x; 1.1081x over previous
import functools

import jax
import jax.numpy as jnp
from jax.experimental import pallas as pl
from jax.experimental.pallas import tpu as pltpu
from jax.experimental.pallas import tpu_sc as plsc

_EMBED_DIM = 64
_NUM_CODES = 1024
_N_TOK = 9216
_TILE = 1024

_NC = 2
_NS = 16
_NW = _NC * _NS
_BPW = _N_TOK // _NW


def _idx_body(z_ref, w_ref, idx_ref):
    zt = z_ref[...]
    zn = zt / jnp.maximum(
        jnp.sqrt(jnp.sum(zt * zt, axis=1, keepdims=True)), 1e-12)
    w = w_ref[...]
    wn = w / jnp.maximum(
        jnp.sqrt(jnp.sum(w * w, axis=1, keepdims=True)), 1e-12)
    scores = jax.lax.dot_general(
        zn, wn, (((1,), (1,)), ((), ())),
        preferred_element_type=jnp.float32)
    m = jnp.max(scores, axis=1, keepdims=True)
    ids = jax.lax.broadcasted_iota(jnp.int32, scores.shape, 1)
    idx_ref[...] = jnp.min(
        jnp.where(scores == m, ids, jnp.int32(2**30)), axis=1)


def _gather_body(w_hbm, idx_hbm, out_hbm, idx_v, rows_v, sem):
    wid = jax.lax.axis_index("s") * _NC + jax.lax.axis_index("c")
    base = wid * _BPW
    pltpu.sync_copy(idx_hbm.at[pl.ds(base, _BPW)], idx_v)
    pltpu.async_copy(w_hbm.at[idx_v], rows_v, sem).wait()
    pltpu.sync_copy(rows_v, out_hbm.at[pl.ds(base, _BPW)])


_sc_gather = pl.kernel(
    _gather_body,
    out_type=jax.ShapeDtypeStruct((_N_TOK, _EMBED_DIM), jnp.float32),
    mesh=plsc.VectorSubcoreMesh(
        core_axis_name="c", subcore_axis_name="s",
        num_cores=_NC, num_subcores=_NS),
    scratch_types=[
        pltpu.VMEM((_BPW,), jnp.int32),
        pltpu.VMEM((_BPW, _EMBED_DIM), jnp.float32),
        pltpu.SemaphoreType.DMA,
    ],
    compiler_params=pltpu.CompilerParams(use_tc_tiling_on_sc=False),
)


@jax.jit
def kernel(z, W):
    z2 = z.reshape(-1, _EMBED_DIM)
    idx = pl.pallas_call(
        _idx_body,
        grid=(_N_TOK // _TILE,),
        in_specs=[
            pl.BlockSpec((_TILE, _EMBED_DIM), lambda i: (i, 0)),
            pl.BlockSpec((_NUM_CODES, _EMBED_DIM), lambda i: (0, 0)),
        ],
        out_specs=pl.BlockSpec((_TILE,), lambda i: (i,)),
        out_shape=jax.ShapeDtypeStruct((_N_TOK,), jnp.int32),
    )(z2, W)
    zq = _sc_gather(W, idx)
    return zq.reshape(z.shape)


# speedup vs baseline: 1.4329x; 1.2931x over previous
import functools

import jax
import jax.numpy as jnp
from jax.experimental import pallas as pl
from jax.experimental.pallas import tpu as pltpu
from jax.experimental.pallas import tpu_sc as plsc

_EMBED_DIM = 64
_NUM_CODES = 1024
_B = 16
_S = 576
_N_TOK = _B * _S
_RPS = 8
_TILE = _RPS * _S
_STEPS = _B // _RPS

_NC = 2
_NS = 16
_NW = _NC * _NS
_BPW = _N_TOK // _NW
_WPB = _NW // _B


def _idx_body(z_ref, w_ref, idx_ref, wn_ref):
    @pl.when(pl.program_id(0) == 0)
    def _init():
        w = w_ref[...]
        wn_ref[...] = w / jnp.maximum(
            jnp.sqrt(jnp.sum(w * w, axis=1, keepdims=True)), 1e-12)

    zt = z_ref[...].reshape(_TILE, _EMBED_DIM)
    zn = zt / jnp.maximum(
        jnp.sqrt(jnp.sum(zt * zt, axis=1, keepdims=True)), 1e-12)
    scores = jax.lax.dot_general(
        zn, wn_ref[...], (((1,), (1,)), ((), ())),
        preferred_element_type=jnp.float32)
    m = jnp.max(scores, axis=1, keepdims=True)
    ids = jax.lax.broadcasted_iota(
        jnp.int32, scores.shape, 1).astype(jnp.float32)
    idx = jnp.min(jnp.where(scores == m, ids, jnp.float32(4096.0)), axis=1)
    i = pl.program_id(0)
    idx_ref[pl.ds(i * _RPS, _RPS), :] = (
        idx.astype(jnp.int32).reshape(_RPS, _S))


def _gather_body(w_hbm, idx_hbm, out_hbm, idx_v, rows_v, sem):
    wid = jax.lax.axis_index("s") * _NC + jax.lax.axis_index("c")
    b = wid // _WPB
    col = (wid % _WPB) * _BPW
    pltpu.sync_copy(idx_hbm.at[b, pl.ds(col, _BPW)], idx_v)
    pltpu.async_copy(w_hbm.at[idx_v], rows_v, sem).wait()
    pltpu.sync_copy(rows_v, out_hbm.at[b, pl.ds(col, _BPW)])


_sc_gather = pl.kernel(
    _gather_body,
    out_type=jax.ShapeDtypeStruct((_B, _S, _EMBED_DIM), jnp.float32),
    mesh=plsc.VectorSubcoreMesh(
        core_axis_name="c", subcore_axis_name="s",
        num_cores=_NC, num_subcores=_NS),
    scratch_types=[
        pltpu.VMEM((_BPW,), jnp.int32),
        pltpu.VMEM((_BPW, _EMBED_DIM), jnp.float32),
        pltpu.SemaphoreType.DMA,
    ],
    compiler_params=pltpu.CompilerParams(use_tc_tiling_on_sc=False),
)


@jax.jit
def kernel(z, W):
    idx = pl.pallas_call(
        _idx_body,
        grid=(_STEPS,),
        in_specs=[
            pl.BlockSpec((_RPS, _S, _EMBED_DIM), lambda i: (i, 0, 0)),
            pl.BlockSpec((_NUM_CODES, _EMBED_DIM), lambda i: (0, 0)),
        ],
        out_specs=pl.BlockSpec((_B, _S), lambda i: (0, 0)),
        out_shape=jax.ShapeDtypeStruct((_B, _S), jnp.int32),
        scratch_shapes=[pltpu.VMEM((_NUM_CODES, _EMBED_DIM), jnp.float32)],
    )(z, W)
    return _sc_gather(W, idx)
